# baseline (speedup 1.0000x reference)
import functools

import jax
import jax.numpy as jnp
from jax import lax
from jax.experimental import pallas as pl
from jax.experimental.pallas import tpu as pltpu
from jax.experimental.pallas import tpu_sc as plsc

N = 10000
E = 320000
NC = 2
NS = 16
NW = NC * NS
EPW = E // NW
CH_FOR = {1: 2000, 16: 2500, 32: 1000, 64: 250}

RPT = 624
RLAST = N - 15 * RPT


def _mesh():
  return plsc.VectorSubcoreMesh(
      core_axis_name="c", subcore_axis_name="s", num_cores=NC, num_subcores=NS
  )


_ZERO16 = None


def _make_sc_deg():
  @functools.partial(
      pl.kernel,
      out_type=jax.ShapeDtypeStruct((2 * N,), jnp.float32),
      mesh=_mesh(),
      compiler_params=pltpu.CompilerParams(use_tc_tiling_on_sc=False),
      scratch_types=[
          pltpu.VMEM((EPW // CH_FOR[1], CH_FOR[1]), jnp.int32),
          pltpu.VMEM((CH_FOR[1],), jnp.float32),
          pltpu.VMEM((RLAST,), jnp.float32),
          pltpu.VMEM_SHARED((N,), jnp.float32),
      ],
  )
  def sc_deg(edge_hbm, out_hbm, didx, ones_v, zbuf, acc):
    CH = CH_FOR[1]
    NCH = EPW // CH
    cid = lax.axis_index("c")
    sid = lax.axis_index("s")
    wid = cid * NS + sid
    pltpu.sync_copy(edge_hbm.at[1, wid], didx)
    one16 = jnp.full((16,), 1.0, jnp.float32)
    zero16 = jnp.zeros((16,), jnp.float32)

    def fill_ones(r, carry):
      ones_v[pl.ds(r * 16, 16)] = one16
      return carry

    lax.fori_loop(0, CH // 16, fill_ones, None)
    for i in range(RLAST // 16):
      zbuf[pl.ds(i * 16, 16)] = zero16

    row0 = sid * RPT
    is_last = sid == NS - 1

    @pl.when(jnp.logical_not(is_last))
    def _():
      pltpu.sync_copy(zbuf.at[pl.ds(0, RPT)], acc.at[pl.ds(row0, RPT)])

    @pl.when(is_last)
    def _():
      pltpu.sync_copy(zbuf, acc.at[pl.ds(15 * RPT, RLAST)])

    plsc.subcore_barrier()

    def body(j, carry):
      pltpu.sync_copy(ones_v, acc.at[didx.at[j]], add=True)
      return carry

    lax.fori_loop(0, NCH, body, None)
    plsc.subcore_barrier()

    @pl.when(jnp.logical_not(is_last))
    def _():
      pltpu.sync_copy(acc.at[pl.ds(row0, RPT)], zbuf.at[pl.ds(0, RPT)])
      pltpu.sync_copy(
          zbuf.at[pl.ds(0, RPT)], out_hbm.at[pl.ds(cid * N + row0, RPT)]
      )

    @pl.when(is_last)
    def _():
      pltpu.sync_copy(acc.at[pl.ds(15 * RPT, RLAST)], zbuf)
      pltpu.sync_copy(zbuf, out_hbm.at[pl.ds(cid * N + 15 * RPT, RLAST)])

  return sc_deg


def _make_sc_scatter(D):
  CH = CH_FOR[D]
  NCH = EPW // CH

  @functools.partial(
      pl.kernel,
      out_type=jax.ShapeDtypeStruct((2 * N, D), jnp.float32),
      mesh=_mesh(),
      compiler_params=pltpu.CompilerParams(use_tc_tiling_on_sc=False),
      scratch_types=[
          pltpu.VMEM((NCH, CH), jnp.int32),
          pltpu.VMEM((NCH, CH), jnp.int32),
          pltpu.VMEM((CH, D), jnp.float32),
          pltpu.VMEM((CH, D), jnp.float32),
          pltpu.VMEM((320, D), jnp.float32),
          pltpu.VMEM_SHARED((N, D), jnp.float32),
          pltpu.SemaphoreType.DMA,
          pltpu.SemaphoreType.DMA,
      ],
  )
  def sc_scatter(
      g_hbm, edge_hbm, out_hbm,
      sidx, didx, rows0, rows1, zbuf, acc, sem0, sem1,
  ):
    cid = lax.axis_index("c")
    sid = lax.axis_index("s")
    wid = cid * NS + sid
    pltpu.sync_copy(edge_hbm.at[0, wid], sidx)
    pltpu.sync_copy(edge_hbm.at[1, wid], didx)

    zero16 = jnp.zeros((16,), jnp.float32)

    def fill_row(r, carry):
      for k in range(D // 16):
        zbuf[r, pl.ds(k * 16, 16)] = zero16
      return carry

    lax.fori_loop(0, 320, fill_row, None)

    row0 = sid * RPT
    is_last = sid == NS - 1

    @pl.when(jnp.logical_not(is_last))
    def _():
      pltpu.sync_copy(zbuf, acc.at[pl.ds(row0, 320)])
      pltpu.sync_copy(
          zbuf.at[pl.ds(0, RPT - 320)], acc.at[pl.ds(row0 + 320, RPT - 320)]
      )

    @pl.when(is_last)
    def _():
      pltpu.sync_copy(zbuf, acc.at[pl.ds(15 * RPT, 320)])
      pltpu.sync_copy(zbuf, acc.at[pl.ds(15 * RPT + 320, RLAST - 320)])

    plsc.subcore_barrier()

    pltpu.async_copy(g_hbm.at[sidx.at[0]], rows0, sem0)

    def body(i, carry):
      j = i * 2
      pltpu.async_copy(g_hbm.at[sidx.at[j + 1]], rows1, sem1)
      pltpu.make_async_copy(g_hbm.at[sidx.at[j]], rows0, sem0).wait()
      pltpu.sync_copy(rows0, acc.at[didx.at[j]], add=True)

      @pl.when(j + 2 < NCH)
      def _():
        pltpu.async_copy(g_hbm.at[sidx.at[j + 2]], rows0, sem0)

      pltpu.make_async_copy(g_hbm.at[sidx.at[j + 1]], rows1, sem1).wait()
      pltpu.sync_copy(rows1, acc.at[didx.at[j + 1]], add=True)
      return carry

    lax.fori_loop(0, NCH // 2, body, None)
    plsc.subcore_barrier()

    @pl.when(jnp.logical_not(is_last))
    def _():
      pltpu.sync_copy(acc.at[pl.ds(row0, 320)], zbuf)
      pltpu.sync_copy(zbuf, out_hbm.at[pl.ds(cid * N + row0, 320)])
      pltpu.sync_copy(
          acc.at[pl.ds(row0 + 320, RPT - 320)], zbuf.at[pl.ds(0, RPT - 320)]
      )
      pltpu.sync_copy(
          zbuf.at[pl.ds(0, RPT - 320)],
          out_hbm.at[pl.ds(cid * N + row0 + 320, RPT - 320)],
      )

    @pl.when(is_last)
    def _():
      pltpu.sync_copy(acc.at[pl.ds(15 * RPT, 320)], zbuf)
      pltpu.sync_copy(zbuf, out_hbm.at[pl.ds(cid * N + 15 * RPT, 320)])
      pltpu.sync_copy(acc.at[pl.ds(15 * RPT + 320, RLAST - 320)], zbuf)
      pltpu.sync_copy(
          zbuf, out_hbm.at[pl.ds(cid * N + 15 * RPT + 320, RLAST - 320)]
      )

  return sc_scatter


@functools.lru_cache(maxsize=None)
def _sc_deg_cached():
  return _make_sc_deg()


@functools.lru_cache(maxsize=None)
def _sc_scatter_cached(d):
  return _make_sc_scatter(d)


def _sc_deg(*args):
  return _sc_deg_cached()(*args)


def _sc_scatter_call(d, *args):
  return _sc_scatter_cached(d)(*args)


def _elu(p):
  return jnp.where(p > 0, p, jnp.exp(jnp.minimum(p, 0.0)) - 1.0)


BN = 2000


def _tc_prep(x, w1, d0, d1):
  dout = w1.shape[1]

  def body(x_ref, w_ref, d0_ref, d1_ref, g_ref, dis_ref):
    deg = d0_ref[...] + d1_ref[...] + 1.0
    dis = lax.rsqrt(deg)
    dis_ref[...] = dis
    g_ref[...] = (
        jnp.dot(x_ref[...], w_ref[...], preferred_element_type=jnp.float32)
        * dis
    )

  return pl.pallas_call(
      body,
      grid=(N // BN,),
      in_specs=[
          pl.BlockSpec((BN, x.shape[1]), lambda i: (i, 0)),
          pl.BlockSpec(w1.shape, lambda i: (0, 0)),
          pl.BlockSpec((BN, 1), lambda i: (i, 0)),
          pl.BlockSpec((BN, 1), lambda i: (i, 0)),
      ],
      out_specs=[
          pl.BlockSpec((BN, dout), lambda i: (i, 0)),
          pl.BlockSpec((BN, 1), lambda i: (i, 0)),
      ],
      out_shape=[
          jax.ShapeDtypeStruct((N, dout), jnp.float32),
          jax.ShapeDtypeStruct((N, 1), jnp.float32),
      ],
  )(x, w1, d0, d1)


def _tc_mid(acc2, g, dis, b, wn):
  dout = wn.shape[1]
  d_in = g.shape[1]

  def body(a0_ref, a1_ref, g_ref, dis_ref, b_ref, w_ref, o_ref):
    dis = dis_ref[...]
    h = _elu(dis * (a0_ref[...] + a1_ref[...] + g_ref[...]) + b_ref[...])
    o_ref[...] = (
        jnp.dot(h, w_ref[...], preferred_element_type=jnp.float32) * dis
    )

  return pl.pallas_call(
      body,
      grid=(N // BN,),
      in_specs=[
          pl.BlockSpec((BN, d_in), lambda i: (i, 0)),
          pl.BlockSpec((BN, d_in), lambda i: (i + N // BN, 0)),
          pl.BlockSpec((BN, d_in), lambda i: (i, 0)),
          pl.BlockSpec((BN, 1), lambda i: (i, 0)),
          pl.BlockSpec((1, d_in), lambda i: (0, 0)),
          pl.BlockSpec(wn.shape, lambda i: (0, 0)),
      ],
      out_specs=pl.BlockSpec((BN, dout), lambda i: (i, 0)),
      out_shape=jax.ShapeDtypeStruct((N, dout), jnp.float32),
  )(acc2, acc2, g, dis, b, wn)


def _tc_final(acc2, g, dis, b3, wf1, bf1, wf2, bf2, wf3, bf3):
  def body(a0_ref, a1_ref, g_ref, dis_ref, b3_ref, wf1_ref, bf1_ref, wf2_ref,
           bf2_ref, wf3_ref, bf3_ref, o_ref):
    h = _elu(
        dis_ref[...] * (a0_ref[...] + a1_ref[...] + g_ref[...]) + b3_ref[...]
    )
    t = _elu(
        jnp.dot(h, wf1_ref[...], preferred_element_type=jnp.float32)
        + bf1_ref[...]
    )
    t = _elu(
        jnp.dot(t, wf2_ref[...], preferred_element_type=jnp.float32)
        + bf2_ref[...]
    )
    t = (
        jnp.dot(t, wf3_ref[...], preferred_element_type=jnp.float32)
        + bf3_ref[...]
    )
    m = jnp.max(t, axis=1, keepdims=True)
    lse = m + jnp.log(jnp.sum(jnp.exp(t - m), axis=1, keepdims=True))
    o_ref[...] = t - lse

  d_in = g.shape[1]
  return pl.pallas_call(
      body,
      grid=(N // BN,),
      in_specs=[
          pl.BlockSpec((BN, d_in), lambda i: (i, 0)),
          pl.BlockSpec((BN, d_in), lambda i: (i + N // BN, 0)),
          pl.BlockSpec((BN, d_in), lambda i: (i, 0)),
          pl.BlockSpec((BN, 1), lambda i: (i, 0)),
          pl.BlockSpec((1, d_in), lambda i: (0, 0)),
          pl.BlockSpec(wf1.shape, lambda i: (0, 0)),
          pl.BlockSpec((1, wf1.shape[1]), lambda i: (0, 0)),
          pl.BlockSpec(wf2.shape, lambda i: (0, 0)),
          pl.BlockSpec((1, wf2.shape[1]), lambda i: (0, 0)),
          pl.BlockSpec(wf3.shape, lambda i: (0, 0)),
          pl.BlockSpec((1, wf3.shape[1]), lambda i: (0, 0)),
      ],
      out_specs=pl.BlockSpec((BN, 2), lambda i: (i, 0)),
      out_shape=jax.ShapeDtypeStruct((N, 2), jnp.float32),
  )(acc2, acc2, g, dis, b3, wf1, bf1, wf2, bf2, wf3, bf3)


def kernel(x, edge_index, W1, b1, W2, b2, W3, b3, Wf1, bf1, Wf2, bf2, Wf3, bf3):
  ei = edge_index.astype(jnp.int32)
  ev = {d: ei.reshape(2, NW, EPW // CH_FOR[d], CH_FOR[d]) for d in (1, 16, 32, 64)}

  degs = _sc_deg(ev[1])
  d0 = degs[:N].reshape(N, 1)
  d1 = degs[N:].reshape(N, 1)

  g1, dis = _tc_prep(x, W1, d0, d1)

  acc = _sc_scatter_call(16, g1, ev[16])
  g2 = _tc_mid(acc, g1, dis, b1.reshape(1, -1), W2)

  acc = _sc_scatter_call(32, g2, ev[32])
  g3 = _tc_mid(acc, g2, dis, b2.reshape(1, -1), W3)

  acc = _sc_scatter_call(64, g3, ev[64])
  return _tc_final(
      acc, g3, dis, b3.reshape(1, -1),
      Wf1, bf1.reshape(1, -1), Wf2, bf2.reshape(1, -1), Wf3, bf3.reshape(1, -1),
  )


# speedup vs baseline: 1.0161x; 1.0161x over previous
import functools

import jax
import jax.numpy as jnp
from jax import lax
from jax.experimental import pallas as pl
from jax.experimental.pallas import tpu as pltpu
from jax.experimental.pallas import tpu_sc as plsc

N = 10000
E = 320000
NC = 2
NS = 16
NW = NC * NS
EPW = E // NW
CH_FOR = {1: 2000, 16: 2500, 32: 1000, 64: 250}

RPT = 624
RLAST = N - 15 * RPT


def _mesh():
  return plsc.VectorSubcoreMesh(
      core_axis_name="c", subcore_axis_name="s", num_cores=NC, num_subcores=NS
  )


_ZERO16 = None


def _make_sc_deg():
  @functools.partial(
      pl.kernel,
      out_type=jax.ShapeDtypeStruct((2 * N,), jnp.float32),
      mesh=_mesh(),
      compiler_params=pltpu.CompilerParams(use_tc_tiling_on_sc=False),
      scratch_types=[
          pltpu.VMEM((EPW // CH_FOR[1], CH_FOR[1]), jnp.int32),
          pltpu.VMEM((CH_FOR[1],), jnp.float32),
          pltpu.VMEM((RLAST,), jnp.float32),
          pltpu.VMEM_SHARED((N,), jnp.float32),
      ],
  )
  def sc_deg(edge_hbm, out_hbm, didx, ones_v, zbuf, acc):
    CH = CH_FOR[1]
    NCH = EPW // CH
    cid = lax.axis_index("c")
    sid = lax.axis_index("s")
    wid = cid * NS + sid
    pltpu.sync_copy(edge_hbm.at[1, wid], didx)
    one16 = jnp.full((16,), 1.0, jnp.float32)
    zero16 = jnp.zeros((16,), jnp.float32)

    def fill_ones(r, carry):
      ones_v[pl.ds(r * 16, 16)] = one16
      return carry

    lax.fori_loop(0, CH // 16, fill_ones, None)
    for i in range(RLAST // 16):
      zbuf[pl.ds(i * 16, 16)] = zero16

    row0 = sid * RPT
    is_last = sid == NS - 1

    @pl.when(jnp.logical_not(is_last))
    def _():
      pltpu.sync_copy(zbuf.at[pl.ds(0, RPT)], acc.at[pl.ds(row0, RPT)])

    @pl.when(is_last)
    def _():
      pltpu.sync_copy(zbuf, acc.at[pl.ds(15 * RPT, RLAST)])

    plsc.subcore_barrier()

    def body(j, carry):
      pltpu.sync_copy(ones_v, acc.at[didx.at[j]], add=True)
      return carry

    lax.fori_loop(0, NCH, body, None)
    plsc.subcore_barrier()

    @pl.when(jnp.logical_not(is_last))
    def _():
      pltpu.sync_copy(acc.at[pl.ds(row0, RPT)], zbuf.at[pl.ds(0, RPT)])
      pltpu.sync_copy(
          zbuf.at[pl.ds(0, RPT)], out_hbm.at[pl.ds(cid * N + row0, RPT)]
      )

    @pl.when(is_last)
    def _():
      pltpu.sync_copy(acc.at[pl.ds(15 * RPT, RLAST)], zbuf)
      pltpu.sync_copy(zbuf, out_hbm.at[pl.ds(cid * N + 15 * RPT, RLAST)])

  return sc_deg


def _make_sc_scatter(D):
  CH = CH_FOR[D]
  NCH = EPW // CH

  @functools.partial(
      pl.kernel,
      out_type=jax.ShapeDtypeStruct((2 * N, D), jnp.float32),
      mesh=_mesh(),
      compiler_params=pltpu.CompilerParams(use_tc_tiling_on_sc=False),
      scratch_types=[
          pltpu.VMEM((NCH, CH), jnp.int32),
          pltpu.VMEM((NCH, CH), jnp.int32),
          pltpu.VMEM((CH, D), jnp.float32),
          pltpu.VMEM((CH, D), jnp.float32),
          pltpu.VMEM((320, D), jnp.float32),
          pltpu.VMEM_SHARED((N, D), jnp.float32),
          pltpu.SemaphoreType.DMA,
          pltpu.SemaphoreType.DMA,
      ],
  )
  def sc_scatter(
      g_hbm, edge_hbm, out_hbm,
      sidx, didx, rows0, rows1, zbuf, acc, sem0, sem1,
  ):
    cid = lax.axis_index("c")
    sid = lax.axis_index("s")
    wid = cid * NS + sid
    pltpu.sync_copy(edge_hbm.at[0, wid], sidx)
    pltpu.sync_copy(edge_hbm.at[1, wid], didx)

    zero16 = jnp.zeros((16,), jnp.float32)

    def fill_row(r, carry):
      for k in range(D // 16):
        zbuf[r, pl.ds(k * 16, 16)] = zero16
      return carry

    lax.fori_loop(0, 320, fill_row, None)

    row0 = sid * RPT
    is_last = sid == NS - 1

    @pl.when(jnp.logical_not(is_last))
    def _():
      pltpu.sync_copy(zbuf, acc.at[pl.ds(row0, 320)])
      pltpu.sync_copy(
          zbuf.at[pl.ds(0, RPT - 320)], acc.at[pl.ds(row0 + 320, RPT - 320)]
      )

    @pl.when(is_last)
    def _():
      pltpu.sync_copy(zbuf, acc.at[pl.ds(15 * RPT, 320)])
      pltpu.sync_copy(zbuf, acc.at[pl.ds(15 * RPT + 320, RLAST - 320)])

    plsc.subcore_barrier()

    pltpu.async_copy(g_hbm.at[sidx.at[0]], rows0, sem0)

    def body(i, carry):
      j = i * 2
      pltpu.async_copy(g_hbm.at[sidx.at[j + 1]], rows1, sem1)
      pltpu.make_async_copy(g_hbm.at[sidx.at[j]], rows0, sem0).wait()
      pltpu.sync_copy(rows0, acc.at[didx.at[j]], add=True)

      @pl.when(j + 2 < NCH)
      def _():
        pltpu.async_copy(g_hbm.at[sidx.at[j + 2]], rows0, sem0)

      pltpu.make_async_copy(g_hbm.at[sidx.at[j + 1]], rows1, sem1).wait()
      pltpu.sync_copy(rows1, acc.at[didx.at[j + 1]], add=True)
      return carry

    lax.fori_loop(0, NCH // 2, body, None)
    plsc.subcore_barrier()

    @pl.when(jnp.logical_not(is_last))
    def _():
      pltpu.sync_copy(acc.at[pl.ds(row0, 320)], zbuf)
      pltpu.sync_copy(zbuf, out_hbm.at[pl.ds(cid * N + row0, 320)])
      pltpu.sync_copy(
          acc.at[pl.ds(row0 + 320, RPT - 320)], zbuf.at[pl.ds(0, RPT - 320)]
      )
      pltpu.sync_copy(
          zbuf.at[pl.ds(0, RPT - 320)],
          out_hbm.at[pl.ds(cid * N + row0 + 320, RPT - 320)],
      )

    @pl.when(is_last)
    def _():
      pltpu.sync_copy(acc.at[pl.ds(15 * RPT, 320)], zbuf)
      pltpu.sync_copy(zbuf, out_hbm.at[pl.ds(cid * N + 15 * RPT, 320)])
      pltpu.sync_copy(acc.at[pl.ds(15 * RPT + 320, RLAST - 320)], zbuf)
      pltpu.sync_copy(
          zbuf, out_hbm.at[pl.ds(cid * N + 15 * RPT + 320, RLAST - 320)]
      )

  return sc_scatter


@functools.lru_cache(maxsize=None)
def _sc_deg_cached():
  return _make_sc_deg()


@functools.lru_cache(maxsize=None)
def _sc_scatter_cached(d):
  return _make_sc_scatter(d)


def _sc_deg(*args):
  return _sc_deg_cached()(*args)


def _sc_scatter_call(d, *args):
  return _sc_scatter_cached(d)(*args)


def _elu(p):
  return jnp.where(p > 0, p, jnp.exp(jnp.minimum(p, 0.0)) - 1.0)


BN = 5000


def _tc_prep(x, w1, d0, d1):
  dout = w1.shape[1]

  def body(x_ref, w_ref, d0_ref, d1_ref, g_ref, dis_ref):
    deg = d0_ref[...] + d1_ref[...] + 1.0
    dis = lax.rsqrt(deg)
    dis_ref[...] = dis
    g_ref[...] = (
        jnp.dot(x_ref[...], w_ref[...], preferred_element_type=jnp.float32)
        * dis
    )

  return pl.pallas_call(
      body,
      grid=(N // BN,),
      in_specs=[
          pl.BlockSpec((BN, x.shape[1]), lambda i: (i, 0)),
          pl.BlockSpec(w1.shape, lambda i: (0, 0)),
          pl.BlockSpec((BN, 1), lambda i: (i, 0)),
          pl.BlockSpec((BN, 1), lambda i: (i, 0)),
      ],
      out_specs=[
          pl.BlockSpec((BN, dout), lambda i: (i, 0)),
          pl.BlockSpec((BN, 1), lambda i: (i, 0)),
      ],
      out_shape=[
          jax.ShapeDtypeStruct((N, dout), jnp.float32),
          jax.ShapeDtypeStruct((N, 1), jnp.float32),
      ],
  )(x, w1, d0, d1)


def _tc_mid(acc2, g, dis, b, wn):
  dout = wn.shape[1]
  d_in = g.shape[1]

  def body(a0_ref, a1_ref, g_ref, dis_ref, b_ref, w_ref, o_ref):
    dis = dis_ref[...]
    h = _elu(dis * (a0_ref[...] + a1_ref[...] + g_ref[...]) + b_ref[...])
    o_ref[...] = (
        jnp.dot(h, w_ref[...], preferred_element_type=jnp.float32) * dis
    )

  return pl.pallas_call(
      body,
      grid=(N // BN,),
      in_specs=[
          pl.BlockSpec((BN, d_in), lambda i: (i, 0)),
          pl.BlockSpec((BN, d_in), lambda i: (i + N // BN, 0)),
          pl.BlockSpec((BN, d_in), lambda i: (i, 0)),
          pl.BlockSpec((BN, 1), lambda i: (i, 0)),
          pl.BlockSpec((1, d_in), lambda i: (0, 0)),
          pl.BlockSpec(wn.shape, lambda i: (0, 0)),
      ],
      out_specs=pl.BlockSpec((BN, dout), lambda i: (i, 0)),
      out_shape=jax.ShapeDtypeStruct((N, dout), jnp.float32),
  )(acc2, acc2, g, dis, b, wn)


def _tc_final(acc2, g, dis, b3, wf1, bf1, wf2, bf2, wf3, bf3):
  def body(a0_ref, a1_ref, g_ref, dis_ref, b3_ref, wf1_ref, bf1_ref, wf2_ref,
           bf2_ref, wf3_ref, bf3_ref, o_ref):
    h = _elu(
        dis_ref[...] * (a0_ref[...] + a1_ref[...] + g_ref[...]) + b3_ref[...]
    )
    t = _elu(
        jnp.dot(h, wf1_ref[...], preferred_element_type=jnp.float32)
        + bf1_ref[...]
    )
    t = _elu(
        jnp.dot(t, wf2_ref[...], preferred_element_type=jnp.float32)
        + bf2_ref[...]
    )
    t = (
        jnp.dot(t, wf3_ref[...], preferred_element_type=jnp.float32)
        + bf3_ref[...]
    )
    m = jnp.max(t, axis=1, keepdims=True)
    lse = m + jnp.log(jnp.sum(jnp.exp(t - m), axis=1, keepdims=True))
    o_ref[...] = t - lse

  d_in = g.shape[1]
  return pl.pallas_call(
      body,
      grid=(N // BN,),
      in_specs=[
          pl.BlockSpec((BN, d_in), lambda i: (i, 0)),
          pl.BlockSpec((BN, d_in), lambda i: (i + N // BN, 0)),
          pl.BlockSpec((BN, d_in), lambda i: (i, 0)),
          pl.BlockSpec((BN, 1), lambda i: (i, 0)),
          pl.BlockSpec((1, d_in), lambda i: (0, 0)),
          pl.BlockSpec(wf1.shape, lambda i: (0, 0)),
          pl.BlockSpec((1, wf1.shape[1]), lambda i: (0, 0)),
          pl.BlockSpec(wf2.shape, lambda i: (0, 0)),
          pl.BlockSpec((1, wf2.shape[1]), lambda i: (0, 0)),
          pl.BlockSpec(wf3.shape, lambda i: (0, 0)),
          pl.BlockSpec((1, wf3.shape[1]), lambda i: (0, 0)),
      ],
      out_specs=pl.BlockSpec((BN, 2), lambda i: (i, 0)),
      out_shape=jax.ShapeDtypeStruct((N, 2), jnp.float32),
  )(acc2, acc2, g, dis, b3, wf1, bf1, wf2, bf2, wf3, bf3)


def kernel(x, edge_index, W1, b1, W2, b2, W3, b3, Wf1, bf1, Wf2, bf2, Wf3, bf3):
  ei = edge_index.astype(jnp.int32)
  ev = {d: ei.reshape(2, NW, EPW // CH_FOR[d], CH_FOR[d]) for d in (1, 16, 32, 64)}

  degs = _sc_deg(ev[1])
  d0 = degs[:N].reshape(N, 1)
  d1 = degs[N:].reshape(N, 1)

  g1, dis = _tc_prep(x, W1, d0, d1)

  acc = _sc_scatter_call(16, g1, ev[16])
  g2 = _tc_mid(acc, g1, dis, b1.reshape(1, -1), W2)

  acc = _sc_scatter_call(32, g2, ev[32])
  g3 = _tc_mid(acc, g2, dis, b2.reshape(1, -1), W3)

  acc = _sc_scatter_call(64, g3, ev[64])
  return _tc_final(
      acc, g3, dis, b3.reshape(1, -1),
      Wf1, bf1.reshape(1, -1), Wf2, bf2.reshape(1, -1), Wf3, bf3.reshape(1, -1),
  )


# speedup vs baseline: 1.0277x; 1.0113x over previous
import functools

import jax
import jax.numpy as jnp
from jax import lax
from jax.experimental import pallas as pl
from jax.experimental.pallas import tpu as pltpu
from jax.experimental.pallas import tpu_sc as plsc

N = 10000
E = 320000
NC = 2
NS = 16
NW = NC * NS
EPW = E // NW
CH = 500
NCH = EPW // CH

RPT = 624
RLAST = N - 15 * RPT


def _mesh():
  return plsc.VectorSubcoreMesh(
      core_axis_name="c", subcore_axis_name="s", num_cores=NC, num_subcores=NS
  )


_ZERO16 = None


def _make_sc_deg():
  @functools.partial(
      pl.kernel,
      out_type=jax.ShapeDtypeStruct((2 * N,), jnp.float32),
      mesh=_mesh(),
      compiler_params=pltpu.CompilerParams(use_tc_tiling_on_sc=False),
      scratch_types=[
          pltpu.VMEM((NCH, CH), jnp.int32),
          pltpu.VMEM((512,), jnp.float32),
          pltpu.VMEM((RLAST,), jnp.float32),
          pltpu.VMEM_SHARED((N,), jnp.float32),
      ],
  )
  def sc_deg(edge_hbm, out_hbm, didx, ones_v, zbuf, acc):
    cid = lax.axis_index("c")
    sid = lax.axis_index("s")
    wid = cid * NS + sid
    pltpu.sync_copy(edge_hbm.at[1, wid], didx)
    one16 = jnp.full((16,), 1.0, jnp.float32)
    zero16 = jnp.zeros((16,), jnp.float32)
    for i in range(512 // 16):
      ones_v[pl.ds(i * 16, 16)] = one16
    for i in range(RLAST // 16):
      zbuf[pl.ds(i * 16, 16)] = zero16

    row0 = sid * RPT
    is_last = sid == NS - 1

    @pl.when(jnp.logical_not(is_last))
    def _():
      pltpu.sync_copy(zbuf.at[pl.ds(0, RPT)], acc.at[pl.ds(row0, RPT)])

    @pl.when(is_last)
    def _():
      pltpu.sync_copy(zbuf, acc.at[pl.ds(15 * RPT, RLAST)])

    plsc.subcore_barrier()

    def body(j, carry):
      pltpu.sync_copy(ones_v.at[pl.ds(0, CH)], acc.at[didx.at[j]], add=True)
      return carry

    lax.fori_loop(0, NCH, body, None)
    plsc.subcore_barrier()

    @pl.when(jnp.logical_not(is_last))
    def _():
      pltpu.sync_copy(acc.at[pl.ds(row0, RPT)], zbuf.at[pl.ds(0, RPT)])
      pltpu.sync_copy(
          zbuf.at[pl.ds(0, RPT)], out_hbm.at[pl.ds(cid * N + row0, RPT)]
      )

    @pl.when(is_last)
    def _():
      pltpu.sync_copy(acc.at[pl.ds(15 * RPT, RLAST)], zbuf)
      pltpu.sync_copy(zbuf, out_hbm.at[pl.ds(cid * N + 15 * RPT, RLAST)])

  return sc_deg


def _make_sc_scatter(D):
  ZR = 320 if D < 64 else 104

  @functools.partial(
      pl.kernel,
      out_type=jax.ShapeDtypeStruct((2 * N, D), jnp.float32),
      mesh=_mesh(),
      compiler_params=pltpu.CompilerParams(use_tc_tiling_on_sc=False),
      scratch_types=[
          pltpu.VMEM((NCH, CH), jnp.int32),
          pltpu.VMEM((NCH, CH), jnp.int32),
          pltpu.VMEM((CH, D), jnp.float32),
          pltpu.VMEM((CH, D), jnp.float32),
          pltpu.VMEM((ZR, D), jnp.float32),
          pltpu.VMEM_SHARED((N, D), jnp.float32),
          pltpu.SemaphoreType.DMA,
          pltpu.SemaphoreType.DMA,
      ],
  )
  def sc_scatter(
      g_hbm, edge_hbm, out_hbm,
      sidx, didx, rows0, rows1, zbuf, acc, sem0, sem1,
  ):
    cid = lax.axis_index("c")
    sid = lax.axis_index("s")
    wid = cid * NS + sid
    pltpu.sync_copy(edge_hbm.at[0, wid], sidx)
    pltpu.sync_copy(edge_hbm.at[1, wid], didx)

    zero16 = jnp.zeros((16,), jnp.float32)

    def fill_row(r, carry):
      for k in range(D // 16):
        zbuf[r, pl.ds(k * 16, 16)] = zero16
      return carry

    lax.fori_loop(0, ZR, fill_row, None)

    row0 = sid * RPT
    is_last = sid == NS - 1

    def init_rows(base, total):
      full, rem = total // ZR, total % ZR
      for k in range(full):
        pltpu.sync_copy(zbuf, acc.at[pl.ds(base + k * ZR, ZR)])
      if rem:
        pltpu.sync_copy(
            zbuf.at[pl.ds(0, rem)], acc.at[pl.ds(base + full * ZR, rem)]
        )

    @pl.when(jnp.logical_not(is_last))
    def _():
      init_rows(row0, RPT)

    @pl.when(is_last)
    def _():
      init_rows(15 * RPT, RLAST)

    plsc.subcore_barrier()

    pltpu.async_copy(g_hbm.at[sidx.at[0]], rows0, sem0)

    def body(i, carry):
      j = i * 2
      pltpu.async_copy(g_hbm.at[sidx.at[j + 1]], rows1, sem1)
      pltpu.make_async_copy(g_hbm.at[sidx.at[j]], rows0, sem0).wait()
      pltpu.sync_copy(rows0, acc.at[didx.at[j]], add=True)

      @pl.when(j + 2 < NCH)
      def _():
        pltpu.async_copy(g_hbm.at[sidx.at[j + 2]], rows0, sem0)

      pltpu.make_async_copy(g_hbm.at[sidx.at[j + 1]], rows1, sem1).wait()
      pltpu.sync_copy(rows1, acc.at[didx.at[j + 1]], add=True)
      return carry

    lax.fori_loop(0, NCH // 2, body, None)
    plsc.subcore_barrier()

    def write_rows(base, total):
      full, rem = total // ZR, total % ZR
      for k in range(full):
        pltpu.sync_copy(acc.at[pl.ds(base + k * ZR, ZR)], zbuf)
        pltpu.sync_copy(zbuf, out_hbm.at[pl.ds(cid * N + base + k * ZR, ZR)])
      if rem:
        pltpu.sync_copy(
            acc.at[pl.ds(base + full * ZR, rem)], zbuf.at[pl.ds(0, rem)]
        )
        pltpu.sync_copy(
            zbuf.at[pl.ds(0, rem)],
            out_hbm.at[pl.ds(cid * N + base + full * ZR, rem)],
        )

    @pl.when(jnp.logical_not(is_last))
    def _():
      write_rows(row0, RPT)

    @pl.when(is_last)
    def _():
      write_rows(15 * RPT, RLAST)

  return sc_scatter


@functools.lru_cache(maxsize=None)
def _sc_deg_cached():
  return _make_sc_deg()


@functools.lru_cache(maxsize=None)
def _sc_scatter_cached(d):
  return _make_sc_scatter(d)


def _sc_deg(*args):
  return _sc_deg_cached()(*args)


def _sc_scatter_call(d, *args):
  return _sc_scatter_cached(d)(*args)


def _elu(p):
  return jnp.where(p > 0, p, jnp.exp(jnp.minimum(p, 0.0)) - 1.0)


BN = 5000


def _tc_prep(x, w1, degs):
  dout = w1.shape[1]

  def body(x_ref, w_ref, d0_ref, d1_ref, g_ref, dis_ref):
    deg = d0_ref[...] + d1_ref[...] + 1.0
    dis = lax.rsqrt(deg)
    dis_ref[...] = dis
    g_ref[...] = (
        jnp.dot(x_ref[...], w_ref[...], preferred_element_type=jnp.float32)
        * dis
    )

  return pl.pallas_call(
      body,
      grid=(N // BN,),
      in_specs=[
          pl.BlockSpec((BN, x.shape[1]), lambda i: (i, 0)),
          pl.BlockSpec(w1.shape, lambda i: (0, 0)),
          pl.BlockSpec((BN, 1), lambda i: (i, 0)),
          pl.BlockSpec((BN, 1), lambda i: (i + N // BN, 0)),
      ],
      out_specs=[
          pl.BlockSpec((BN, dout), lambda i: (i, 0)),
          pl.BlockSpec((BN, 1), lambda i: (i, 0)),
      ],
      out_shape=[
          jax.ShapeDtypeStruct((N, dout), jnp.float32),
          jax.ShapeDtypeStruct((N, 1), jnp.float32),
      ],
  )(x, w1, degs, degs)


def _tc_mid(acc2, g, dis, b, wn):
  dout = wn.shape[1]
  d_in = g.shape[1]

  def body(a0_ref, a1_ref, g_ref, dis_ref, b_ref, w_ref, o_ref):
    dis = dis_ref[...]
    h = _elu(dis * (a0_ref[...] + a1_ref[...] + g_ref[...]) + b_ref[...])
    o_ref[...] = (
        jnp.dot(h, w_ref[...], preferred_element_type=jnp.float32) * dis
    )

  return pl.pallas_call(
      body,
      grid=(N // BN,),
      in_specs=[
          pl.BlockSpec((BN, d_in), lambda i: (i, 0)),
          pl.BlockSpec((BN, d_in), lambda i: (i + N // BN, 0)),
          pl.BlockSpec((BN, d_in), lambda i: (i, 0)),
          pl.BlockSpec((BN, 1), lambda i: (i, 0)),
          pl.BlockSpec((1, d_in), lambda i: (0, 0)),
          pl.BlockSpec(wn.shape, lambda i: (0, 0)),
      ],
      out_specs=pl.BlockSpec((BN, dout), lambda i: (i, 0)),
      out_shape=jax.ShapeDtypeStruct((N, dout), jnp.float32),
  )(acc2, acc2, g, dis, b, wn)


def _tc_final(acc2, g, dis, b3, wf1, bf1, wf2, bf2, wf3, bf3):
  def body(a0_ref, a1_ref, g_ref, dis_ref, b3_ref, wf1_ref, bf1_ref, wf2_ref,
           bf2_ref, wf3_ref, bf3_ref, o_ref):
    h = _elu(
        dis_ref[...] * (a0_ref[...] + a1_ref[...] + g_ref[...]) + b3_ref[...]
    )
    t = _elu(
        jnp.dot(h, wf1_ref[...], preferred_element_type=jnp.float32)
        + bf1_ref[...]
    )
    t = _elu(
        jnp.dot(t, wf2_ref[...], preferred_element_type=jnp.float32)
        + bf2_ref[...]
    )
    t = (
        jnp.dot(t, wf3_ref[...], preferred_element_type=jnp.float32)
        + bf3_ref[...]
    )
    m = jnp.max(t, axis=1, keepdims=True)
    lse = m + jnp.log(jnp.sum(jnp.exp(t - m), axis=1, keepdims=True))
    o_ref[...] = t - lse

  d_in = g.shape[1]
  return pl.pallas_call(
      body,
      grid=(N // BN,),
      in_specs=[
          pl.BlockSpec((BN, d_in), lambda i: (i, 0)),
          pl.BlockSpec((BN, d_in), lambda i: (i + N // BN, 0)),
          pl.BlockSpec((BN, d_in), lambda i: (i, 0)),
          pl.BlockSpec((BN, 1), lambda i: (i, 0)),
          pl.BlockSpec((1, d_in), lambda i: (0, 0)),
          pl.BlockSpec(wf1.shape, lambda i: (0, 0)),
          pl.BlockSpec((1, wf1.shape[1]), lambda i: (0, 0)),
          pl.BlockSpec(wf2.shape, lambda i: (0, 0)),
          pl.BlockSpec((1, wf2.shape[1]), lambda i: (0, 0)),
          pl.BlockSpec(wf3.shape, lambda i: (0, 0)),
          pl.BlockSpec((1, wf3.shape[1]), lambda i: (0, 0)),
      ],
      out_specs=pl.BlockSpec((BN, 2), lambda i: (i, 0)),
      out_shape=jax.ShapeDtypeStruct((N, 2), jnp.float32),
  )(acc2, acc2, g, dis, b3, wf1, bf1, wf2, bf2, wf3, bf3)


def kernel(x, edge_index, W1, b1, W2, b2, W3, b3, Wf1, bf1, Wf2, bf2, Wf3, bf3):
  ev = edge_index.astype(jnp.int32).reshape(2, NW, NCH, CH)

  degs = _sc_deg(ev).reshape(2 * N, 1)

  g1, dis = _tc_prep(x, W1, degs)

  acc = _sc_scatter_call(16, g1, ev)
  g2 = _tc_mid(acc, g1, dis, b1.reshape(1, -1), W2)

  acc = _sc_scatter_call(32, g2, ev)
  g3 = _tc_mid(acc, g2, dis, b2.reshape(1, -1), W3)

  acc = _sc_scatter_call(64, g3, ev)
  return _tc_final(
      acc, g3, dis, b3.reshape(1, -1),
      Wf1, bf1.reshape(1, -1), Wf2, bf2.reshape(1, -1), Wf3, bf3.reshape(1, -1),
  )
